# baseline (speedup 1.0000x reference)
import functools

import jax
import jax.numpy as jnp
from jax.experimental import pallas as pl
from jax.experimental.pallas import tpu as pltpu

OBS_DIM = 4096
HIDDEN = 2048
N_ACTIONS = 4096
N_PIECES = 32
PIECE_VOCAB = 64
BATCH = 128
THRESHOLD = 0.001

H_BLK = 256
A_BLK = 512
N_A_STEPS = HIDDEN // H_BLK
N_B_STEPS = N_ACTIONS // A_BLK


def _fused_kernel(obs_ref, pid_ref, w1_ref, b1_ref, pemb_ref,
                  w2_ref, b2_ref, legal_ref, g_ref,
                  lp_ref, act_ref, h_ref, pe_ref):
    t = pl.program_id(0)

    @pl.when(t == 0)
    def _pe():
        ids = pid_ref[...]
        iota = jax.lax.broadcasted_iota(
            jnp.int32, (BATCH, N_PIECES, PIECE_VOCAB), 2)
        counts = jnp.sum((ids[:, :, None] == iota).astype(jnp.float32), axis=1)
        pe_ref[...] = jnp.dot(counts, pemb_ref[...],
                              preferred_element_type=jnp.float32,
                              precision=jax.lax.Precision.HIGHEST)

    @pl.when(t < N_A_STEPS)
    def _phase_a():
        acc = jnp.dot(obs_ref[...], w1_ref[...].astype(jnp.bfloat16),
                      preferred_element_type=jnp.float32)
        hs = jnp.maximum(acc + b1_ref[...] + pe_ref[:, pl.ds(t * H_BLK, H_BLK)],
                         0.0)
        h_ref[:, pl.ds(t * H_BLK, H_BLK)] = hs.astype(jnp.bfloat16)

    @pl.when(t >= N_A_STEPS)
    def _phase_b():
        i = t - N_A_STEPS
        blk = jnp.dot(h_ref[...], w2_ref[...].astype(jnp.bfloat16),
                      preferred_element_type=jnp.float32)
        blk = blk + b2_ref[...]
        blk = jnp.where(legal_ref[...] > 0, blk, jnp.float32(-1e9))
        lp_ref[:, pl.ds(i * A_BLK, A_BLK)] = blk

    @pl.when(t == N_A_STEPS + N_B_STEPS - 1)
    def _finalize():
        masked = lp_ref[...]
        m = jnp.max(masked, axis=1, keepdims=True)
        shifted = masked - m
        lse = jnp.log(jnp.sum(jnp.exp(shifted), axis=1, keepdims=True))
        log_probs = shifted - lse
        lp_ref[...] = log_probs
        probs = jnp.exp(log_probs)
        probs = jnp.where(probs > THRESHOLD, probs, 0.0)
        probs = probs / jnp.sum(probs, axis=1, keepdims=True)
        scores = jnp.log(jnp.clip(probs, 1e-30, None)) + g_ref[...]
        smax = jnp.max(scores, axis=1, keepdims=True)
        idx = jax.lax.broadcasted_iota(jnp.int32, (BATCH, N_ACTIONS), 1)
        cand = jnp.where(scores == smax, idx, N_ACTIONS)
        act_ref[0, :] = jnp.min(cand, axis=1)


@functools.partial(jax.jit, static_argnames=("interpret",))
def kernel(observations, piece_ids, legal_actions, W1, b1, W2, b2, piece_emb,
           interpret=False):
    piece_ids = piece_ids.astype(jnp.int32)
    obs_bf = observations.astype(jnp.bfloat16)
    b1_2d = b1.reshape(1, HIDDEN)
    b2_2d = b2.reshape(1, N_ACTIONS)
    gumbel = jax.random.gumbel(jax.random.key(42), (BATCH, N_ACTIONS),
                               jnp.float32)

    a_steps = N_A_STEPS

    log_probs, action = pl.pallas_call(
        _fused_kernel,
        grid=(N_A_STEPS + N_B_STEPS,),
        in_specs=[
            pl.BlockSpec((BATCH, OBS_DIM), lambda t: (0, 0)),
            pl.BlockSpec((BATCH, N_PIECES), lambda t: (0, 0)),
            pl.BlockSpec((OBS_DIM, H_BLK),
                         lambda t: (0, jnp.minimum(t, N_A_STEPS - 1))),
            pl.BlockSpec((1, H_BLK),
                         lambda t: (0, jnp.minimum(t, N_A_STEPS - 1))),
            pl.BlockSpec((PIECE_VOCAB, HIDDEN), lambda t: (0, 0)),
            pl.BlockSpec((HIDDEN, A_BLK),
                         lambda t: (0, jnp.clip(t - a_steps, 0, N_B_STEPS - 1))),
            pl.BlockSpec((1, A_BLK),
                         lambda t: (0, jnp.clip(t - a_steps, 0, N_B_STEPS - 1))),
            pl.BlockSpec((BATCH, A_BLK),
                         lambda t: (0, jnp.clip(t - a_steps, 0, N_B_STEPS - 1))),
            pl.BlockSpec((BATCH, N_ACTIONS), lambda t: (0, 0)),
        ],
        out_specs=[
            pl.BlockSpec((BATCH, N_ACTIONS), lambda t: (0, 0)),
            pl.BlockSpec((1, BATCH), lambda t: (0, 0)),
        ],
        out_shape=[
            jax.ShapeDtypeStruct((BATCH, N_ACTIONS), jnp.float32),
            jax.ShapeDtypeStruct((1, BATCH), jnp.int32),
        ],
        scratch_shapes=[
            pltpu.VMEM((BATCH, HIDDEN), jnp.bfloat16),
            pltpu.VMEM((BATCH, HIDDEN), jnp.float32),
        ],
        interpret=interpret,
    )(obs_bf, piece_ids, W1, b1_2d, piece_emb,
      W2, b2_2d, legal_actions, gumbel)

    return (log_probs, action.reshape(BATCH))


# speedup vs baseline: 1.2617x; 1.2617x over previous
import functools

import jax
import jax.numpy as jnp
import numpy as np
from jax.experimental import pallas as pl
from jax.experimental.pallas import tpu as pltpu

OBS_DIM = 4096
HIDDEN = 2048
N_ACTIONS = 4096
N_PIECES = 32
PIECE_VOCAB = 64
BATCH = 128
THRESHOLD = 0.001

H_BLK = 512
A_BLK = 1024
N_A_STEPS = HIDDEN // H_BLK
N_B_STEPS = N_ACTIONS // A_BLK
N_STEPS = N_A_STEPS + N_B_STEPS
G_ROWS = BATCH // N_STEPS

KEY0 = np.uint32(0)
KEY1 = np.uint32(42)


def _gumbel_rows(row0, n_rows):
    u32 = jnp.uint32
    lin = ((jax.lax.broadcasted_iota(jnp.int32, (n_rows, N_ACTIONS), 0) + row0)
           * N_ACTIONS
           + jax.lax.broadcasted_iota(jnp.int32, (n_rows, N_ACTIONS), 1))
    x1 = lin.astype(u32)
    x0 = jnp.zeros_like(x1)
    ks0 = u32(KEY0)
    ks1 = u32(KEY1)
    ks2 = u32(KEY0 ^ KEY1 ^ np.uint32(0x1BD11BDA))
    ks = (ks0, ks1, ks2)
    rotations = ((13, 15, 26, 6), (17, 29, 16, 24))

    def rotl(x, r):
        return (x << u32(r)) | (x >> u32(32 - r))

    x0 = x0 + ks[0]
    x1 = x1 + ks[1]
    for i in range(5):
        for r in rotations[i % 2]:
            x0 = x0 + x1
            x1 = rotl(x1, r)
            x1 = x1 ^ x0
        x0 = x0 + ks[(i + 1) % 3]
        x1 = x1 + ks[(i + 2) % 3] + u32(i + 1)
    bits = x0 ^ x1
    fb = (bits >> u32(9)) | u32(0x3F800000)
    floats = jax.lax.bitcast_convert_type(fb, jnp.float32) - jnp.float32(1.0)
    tiny = jnp.float32(np.finfo(np.float32).tiny)
    u = jnp.maximum(tiny, floats * (jnp.float32(1.0) - tiny) + tiny)
    return -jnp.log(-jnp.log(u))


def _fused_kernel(obs_ref, pid_ref, w1_ref, b1_ref, pemb_ref,
                  w2_ref, b2_ref, legal_ref,
                  lp_ref, act_ref, obs_bf_ref, h_ref, pe_ref, g_ref):
    t = pl.program_id(0)

    g_ref[pl.ds(t * G_ROWS, G_ROWS), :] = _gumbel_rows(t * G_ROWS, G_ROWS)

    @pl.when(t == 0)
    def _prep():
        obs_bf_ref[...] = obs_ref[...].astype(jnp.bfloat16)
        ids = pid_ref[...]
        iota = jax.lax.broadcasted_iota(
            jnp.int32, (BATCH, N_PIECES, PIECE_VOCAB), 2)
        counts = jnp.sum((ids[:, :, None] == iota).astype(jnp.float32), axis=1)
        pe_ref[...] = jnp.dot(counts, pemb_ref[...],
                              preferred_element_type=jnp.float32,
                              precision=jax.lax.Precision.HIGHEST)

    @pl.when(t < N_A_STEPS)
    def _phase_a():
        acc = jnp.dot(obs_bf_ref[...], w1_ref[...].astype(jnp.bfloat16),
                      preferred_element_type=jnp.float32)
        hs = jnp.maximum(acc + b1_ref[...] + pe_ref[:, pl.ds(t * H_BLK, H_BLK)],
                         0.0)
        h_ref[:, pl.ds(t * H_BLK, H_BLK)] = hs.astype(jnp.bfloat16)

    @pl.when(t >= N_A_STEPS)
    def _phase_b():
        i = t - N_A_STEPS
        blk = jnp.dot(h_ref[...], w2_ref[...].astype(jnp.bfloat16),
                      preferred_element_type=jnp.float32)
        blk = blk + b2_ref[...]
        blk = jnp.where(legal_ref[...] > 0, blk, jnp.float32(-1e9))
        lp_ref[:, pl.ds(i * A_BLK, A_BLK)] = blk

    @pl.when(t == N_STEPS - 1)
    def _finalize():
        masked = lp_ref[...]
        m = jnp.max(masked, axis=1, keepdims=True)
        shifted = masked - m
        lse = jnp.log(jnp.sum(jnp.exp(shifted), axis=1, keepdims=True))
        log_probs = shifted - lse
        lp_ref[...] = log_probs
        probs = jnp.exp(log_probs)
        probs = jnp.where(probs > THRESHOLD, probs, 0.0)
        probs = probs / jnp.sum(probs, axis=1, keepdims=True)
        scores = jnp.log(jnp.clip(probs, 1e-30, None)) + g_ref[...]
        smax = jnp.max(scores, axis=1, keepdims=True)
        idx = jax.lax.broadcasted_iota(jnp.int32, (BATCH, N_ACTIONS), 1)
        cand = jnp.where(scores == smax, idx, N_ACTIONS)
        act_ref[0, :] = jnp.min(cand, axis=1)


@functools.partial(jax.jit, static_argnames=("interpret",))
def kernel(observations, piece_ids, legal_actions, W1, b1, W2, b2, piece_emb,
           interpret=False):
    piece_ids = piece_ids.astype(jnp.int32)
    b1_2d = b1.reshape(1, HIDDEN)
    b2_2d = b2.reshape(1, N_ACTIONS)

    a_steps = N_A_STEPS

    log_probs, action = pl.pallas_call(
        _fused_kernel,
        grid=(N_STEPS,),
        in_specs=[
            pl.BlockSpec((BATCH, OBS_DIM), lambda t: (0, 0)),
            pl.BlockSpec((BATCH, N_PIECES), lambda t: (0, 0)),
            pl.BlockSpec((OBS_DIM, H_BLK),
                         lambda t: (0, jnp.minimum(t, N_A_STEPS - 1))),
            pl.BlockSpec((1, H_BLK),
                         lambda t: (0, jnp.minimum(t, N_A_STEPS - 1))),
            pl.BlockSpec((PIECE_VOCAB, HIDDEN), lambda t: (0, 0)),
            pl.BlockSpec((HIDDEN, A_BLK),
                         lambda t: (0, jnp.clip(t - a_steps, 0, N_B_STEPS - 1))),
            pl.BlockSpec((1, A_BLK),
                         lambda t: (0, jnp.clip(t - a_steps, 0, N_B_STEPS - 1))),
            pl.BlockSpec((BATCH, A_BLK),
                         lambda t: (0, jnp.clip(t - a_steps, 0, N_B_STEPS - 1))),
        ],
        out_specs=[
            pl.BlockSpec((BATCH, N_ACTIONS), lambda t: (0, 0)),
            pl.BlockSpec((1, BATCH), lambda t: (0, 0)),
        ],
        out_shape=[
            jax.ShapeDtypeStruct((BATCH, N_ACTIONS), jnp.float32),
            jax.ShapeDtypeStruct((1, BATCH), jnp.int32),
        ],
        scratch_shapes=[
            pltpu.VMEM((BATCH, OBS_DIM), jnp.bfloat16),
            pltpu.VMEM((BATCH, HIDDEN), jnp.bfloat16),
            pltpu.VMEM((BATCH, HIDDEN), jnp.float32),
            pltpu.VMEM((BATCH, N_ACTIONS), jnp.float32),
        ],
        interpret=interpret,
    )(observations, piece_ids, W1, b1_2d, piece_emb,
      W2, b2_2d, legal_actions)

    return (log_probs, action.reshape(BATCH))
